# baseline (speedup 1.0000x reference)
import jax
import jax.numpy as jnp
from jax.experimental import pallas as pl
from jax.experimental.pallas import tpu as pltpu

CHUNK = 1024
NBUF = 3
NOUT = 2


def _gating_kernel(x_hbm, w_ref, b_ref, out_hbm, bufs, wbuf, obufs, isems, osems):
    nchunks = x_hbm.shape[0] // CHUNK
    b = b_ref[...]

    def in_copy(chunk):
        slot = chunk % NBUF
        return pltpu.make_async_copy(
            x_hbm.at[pl.ds(chunk * CHUNK, CHUNK), :],
            bufs.at[slot],
            isems.at[slot],
        )

    def out_copy(chunk):
        slot = chunk % NOUT
        return pltpu.make_async_copy(
            obufs.at[slot],
            out_hbm.at[pl.ds(chunk * CHUNK, CHUNK), :],
            osems.at[slot],
        )

    for c in range(min(NBUF - 1, nchunks)):
        in_copy(c).start()
    wbuf[...] = w_ref[...].astype(jnp.bfloat16)
    for c in range(nchunks):
        if c + NBUF - 1 < nchunks:
            in_copy(c + NBUF - 1).start()
        in_copy(c).wait()
        if c >= NOUT:
            out_copy(c - NOUT).wait()
        xb = bufs[c % NBUF].astype(jnp.bfloat16)
        logits = jax.lax.dot_general(
            xb, wbuf[...], (((1,), (1,)), ((), ())),
            preferred_element_type=jnp.float32,
        )
        logits = logits + b
        m = jnp.max(logits, axis=-1, keepdims=True)
        e = jnp.exp(logits - m)
        obufs[c % NOUT] = e / jnp.sum(e, axis=-1, keepdims=True)
        out_copy(c).start()
    for c in range(max(nchunks - NOUT, 0), nchunks):
        out_copy(c).wait()


def kernel(x, W, b):
    tokens, dim = x.shape
    experts = W.shape[0]
    b2 = b.reshape(1, experts)
    return pl.pallas_call(
        _gating_kernel,
        in_specs=[
            pl.BlockSpec(memory_space=pltpu.MemorySpace.HBM),
            pl.BlockSpec((experts, dim), lambda: (0, 0)),
            pl.BlockSpec((1, experts), lambda: (0, 0)),
        ],
        out_specs=pl.BlockSpec(memory_space=pltpu.MemorySpace.HBM),
        out_shape=jax.ShapeDtypeStruct((tokens, experts), jnp.float32),
        scratch_shapes=[
            pltpu.VMEM((NBUF, CHUNK, dim), jnp.float32),
            pltpu.VMEM((experts, dim), jnp.bfloat16),
            pltpu.VMEM((NOUT, CHUNK, experts), jnp.float32),
            pltpu.SemaphoreType.DMA((NBUF,)),
            pltpu.SemaphoreType.DMA((NOUT,)),
        ],
    )(x, W, b2)


# speedup vs baseline: 1.1379x; 1.1379x over previous
import jax
import jax.numpy as jnp
from jax.experimental import pallas as pl

TOK_BLOCK = 1024


def _gating_kernel(x_ref, w_ref, b_ref, out_ref):
    xb = x_ref[...].astype(jnp.bfloat16)
    wb = w_ref[...].astype(jnp.bfloat16)
    logits = jax.lax.dot_general(
        xb, wb, (((1,), (1,)), ((), ())),
        preferred_element_type=jnp.float32,
    )
    logits = logits + b_ref[...]
    m = jnp.max(logits, axis=-1, keepdims=True)
    e = jnp.exp(logits - m)
    probs = e / jnp.sum(e, axis=-1, keepdims=True)
    out_ref[...] = probs.T


def kernel(x, W, b):
    tokens, dim = x.shape
    experts = W.shape[0]
    b2 = b.reshape(1, experts)
    out_t = pl.pallas_call(
        _gating_kernel,
        grid=(tokens // TOK_BLOCK,),
        in_specs=[
            pl.BlockSpec((TOK_BLOCK, dim), lambda i: (i, 0)),
            pl.BlockSpec((experts, dim), lambda i: (0, 0)),
            pl.BlockSpec((1, experts), lambda i: (0, 0)),
        ],
        out_specs=pl.BlockSpec((experts, TOK_BLOCK), lambda i: (0, i)),
        out_shape=jax.ShapeDtypeStruct((experts, tokens), jnp.float32),
    )(x, W, b2)
    return out_t.T


# speedup vs baseline: 1.1486x; 1.0095x over previous
import jax
import jax.numpy as jnp
from jax.experimental import pallas as pl
from jax.experimental.pallas import tpu as pltpu

TOK_BLOCK = 1024


def _gating_kernel(x_ref, w_ref, b_ref, out_ref, wbuf):
    @pl.when(pl.program_id(0) == 0)
    def _():
        wbuf[...] = w_ref[...].astype(jnp.bfloat16)

    xb = x_ref[...].astype(jnp.bfloat16)
    logits = jax.lax.dot_general(
        xb, wbuf[...], (((1,), (1,)), ((), ())),
        preferred_element_type=jnp.float32,
    )
    logits = logits + b_ref[...]
    m = jnp.max(logits, axis=-1, keepdims=True)
    e = jnp.exp(logits - m)
    probs = e / jnp.sum(e, axis=-1, keepdims=True)
    out_ref[...] = probs.T


def kernel(x, W, b):
    tokens, dim = x.shape
    experts = W.shape[0]
    b2 = b.reshape(1, experts)
    out_t = pl.pallas_call(
        _gating_kernel,
        grid=(tokens // TOK_BLOCK,),
        in_specs=[
            pl.BlockSpec((TOK_BLOCK, dim), lambda i: (i, 0)),
            pl.BlockSpec((experts, dim), lambda i: (0, 0)),
            pl.BlockSpec((1, experts), lambda i: (0, 0)),
        ],
        out_specs=pl.BlockSpec((experts, TOK_BLOCK), lambda i: (0, i)),
        out_shape=jax.ShapeDtypeStruct((experts, tokens), jnp.float32),
        scratch_shapes=[pltpu.VMEM((experts, dim), jnp.bfloat16)],
    )(x, W, b2)
    return out_t.T
